# Initial kernel scaffold; baseline (speedup 1.0000x reference)
import jax
import jax.numpy as jnp
from jax.experimental import pallas as pl


def kernel(x, b, W_enc, b_enc, W_dec, miss_counts):
    raise NotImplementedError("write your pallas kernel here")


# speedup vs baseline: 12.1232x; 12.1232x over previous
import jax
import jax.numpy as jnp
from jax.experimental import pallas as pl
from jax.experimental.pallas import tpu as pltpu

N_IN = 768
N_LAT = 16384
TOPK = 64
ROWS = 4096

B1, L1 = 512, 2048
B2 = 64
B3, L3 = 1024, 2048


def _enc_body(x_ref, b_ref, w_ref, benc_ref, out_ref):
    xc = x_ref[...] - b_ref[...]
    out_ref[...] = jax.lax.dot_general(
        xc, w_ref[...], (((1,), (1,)), ((), ())),
        preferred_element_type=jnp.float32,
    ) + benc_ref[...]


def _topk_body(lg_ref, alpha_ref, mask_ref):
    v = lg_ref[...]
    u = jax.lax.bitcast_convert_type(v, jnp.int32)
    m = jnp.where(u >= 0, u, u ^ jnp.int32(0x7FFFFFFF))

    cnt0 = jnp.sum((m >= 0).astype(jnp.int32), axis=1, keepdims=True)
    t = jnp.where(cnt0 >= TOPK, jnp.int32(0), jnp.int32(-0x80000000))
    for bit in range(30, -1, -1):
        cand = t | jnp.int32(1 << bit)
        cnt = jnp.sum((m >= cand).astype(jnp.int32), axis=1, keepdims=True)
        t = jnp.where(cnt >= TOPK, cand, t)

    sel = m >= t
    alpha_ref[...] = jnp.where(sel, v, 0.0)
    mask_ref[...] = sel & (v != 0.0)


def _dec_body(alpha_ref, wd_ref, b_ref, out_ref):
    k = pl.program_id(1)

    @pl.when(k == 0)
    def _init():
        out_ref[...] = jnp.broadcast_to(b_ref[...], (B3, N_IN))

    out_ref[...] += jax.lax.dot_general(
        alpha_ref[...], wd_ref[...], (((1,), (1,)), ((), ())),
        preferred_element_type=jnp.float32,
    )


@jax.jit
def kernel(x, b, W_enc, b_enc, W_dec, miss_counts):
    del miss_counts
    b2 = b.reshape(1, N_IN)
    benc2 = b_enc.reshape(1, N_LAT)

    logits = pl.pallas_call(
        _enc_body,
        grid=(N_LAT // L1, ROWS // B1),
        in_specs=[
            pl.BlockSpec((B1, N_IN), lambda l, r: (r, 0)),
            pl.BlockSpec((1, N_IN), lambda l, r: (0, 0)),
            pl.BlockSpec((L1, N_IN), lambda l, r: (l, 0)),
            pl.BlockSpec((1, L1), lambda l, r: (0, l)),
        ],
        out_specs=pl.BlockSpec((B1, L1), lambda l, r: (r, l)),
        out_shape=jax.ShapeDtypeStruct((ROWS, N_LAT), jnp.float32),
        compiler_params=pltpu.CompilerParams(
            dimension_semantics=("arbitrary", "arbitrary"),
        ),
    )(x, b2, W_enc, benc2)

    alpha, mask = pl.pallas_call(
        _topk_body,
        grid=(ROWS // B2,),
        in_specs=[pl.BlockSpec((B2, N_LAT), lambda r: (r, 0))],
        out_specs=[
            pl.BlockSpec((B2, N_LAT), lambda r: (r, 0)),
            pl.BlockSpec((B2, N_LAT), lambda r: (r, 0)),
        ],
        out_shape=[
            jax.ShapeDtypeStruct((ROWS, N_LAT), jnp.float32),
            jax.ShapeDtypeStruct((ROWS, N_LAT), jnp.bool_),
        ],
        compiler_params=pltpu.CompilerParams(
            dimension_semantics=("arbitrary",),
        ),
    )(logits)

    xhat = pl.pallas_call(
        _dec_body,
        grid=(ROWS // B3, N_LAT // L3),
        in_specs=[
            pl.BlockSpec((B3, L3), lambda r, k: (r, k)),
            pl.BlockSpec((N_IN, L3), lambda r, k: (0, k)),
            pl.BlockSpec((1, N_IN), lambda r, k: (0, 0)),
        ],
        out_specs=pl.BlockSpec((B3, N_IN), lambda r, k: (r, 0)),
        out_shape=jax.ShapeDtypeStruct((ROWS, N_IN), jnp.float32),
        compiler_params=pltpu.CompilerParams(
            dimension_semantics=("arbitrary", "arbitrary"),
        ),
    )(alpha, W_dec, b2)

    return (xhat, alpha, mask)
